# Initial kernel scaffold; baseline (speedup 1.0000x reference)
import jax
import jax.numpy as jnp
from jax.experimental import pallas as pl


def kernel(batch, x, edge_index, edge_attr, atom_emb, bond_emb, W1, b1, gamma1, beta1, W2, b2, gamma2, beta2):
    raise NotImplementedError("write your pallas kernel here")


# speedup vs baseline: 2.4856x; 2.4856x over previous
import functools

import jax
import jax.numpy as jnp
from jax import lax
from jax.experimental import pallas as pl
from jax.experimental.pallas import tpu as pltpu
from jax.experimental.pallas import tpu_sc as plsc

N = 10000
E = 320000
D = 128
L = 5
NG = 128
ATOM_F = 9
BOND_F = 3
ATOM_VOCAB = 100
BOND_VOCAB = 5

NC = 2
NS = 16
NW = NC * NS
KI = 128
KN = 64
NP = 10240
CHK_N = NP // NW
NSUB_N = CHK_N // KN
EP = 327680
EPW = EP // NW
KE = KI
NCHK = EPW // KE
ROWS_PER_S = NP // NS

_mesh = plsc.VectorSubcoreMesh(core_axis_name="c", subcore_axis_name="s")


@functools.partial(
    pl.kernel,
    out_type=(
        jax.ShapeDtypeStruct((NP, D), jnp.float32),
        jax.ShapeDtypeStruct((EP,), jnp.int32),
    ),
    mesh=_mesh,
    scratch_types=[
        pltpu.VMEM((KN,), jnp.int32),
        pltpu.VMEM((CHK_N, D), jnp.float32),
        pltpu.VMEM((EPW,), jnp.int32),
        pltpu.VMEM((EPW,), jnp.int32),
        pltpu.VMEM((EPW,), jnp.int32),
        pltpu.VMEM((EPW,), jnp.int32),
        pltpu.SemaphoreType.DMA,
    ],
)
def _sc_encode(xT_h, eaT_h, atomf_h, h0_h, code_h,
               idx_v, rows_v, ea0_v, ea1_v, ea2_v, ecode_v, sem):
    c = lax.axis_index("c")
    s = lax.axis_index("s")
    wid = s * NC + c

    base_n = wid * CHK_N
    for f in range(ATOM_F):
        for k in range(NSUB_N):
            pltpu.sync_copy(xT_h.at[pl.ds(f * NP + base_n + k * KN, KN)],
                            idx_v)
            if f > 0:
                def _shift(i, _, f=f):
                    sl = pl.ds(i * 16, 16)
                    idx_v[sl] = idx_v[sl] + f * ATOM_VOCAB
                    return 0
                lax.fori_loop(0, KN // 16, _shift, 0)
            pltpu.async_copy(atomf_h.at[idx_v], rows_v.at[pl.ds(k * KN, KN)],
                             sem, add=(f > 0)).wait()
    pltpu.sync_copy(rows_v, h0_h.at[pl.ds(base_n, CHK_N)])

    base_e = wid * EPW
    pltpu.sync_copy(eaT_h.at[pl.ds(base_e, EPW)], ea0_v)
    pltpu.sync_copy(eaT_h.at[pl.ds(EP + base_e, EPW)], ea1_v)
    pltpu.sync_copy(eaT_h.at[pl.ds(2 * EP + base_e, EPW)], ea2_v)

    def _code(i, _):
        sl = pl.ds(i * 16, 16)
        ecode_v[sl] = (ea0_v[sl] * (BOND_VOCAB * BOND_VOCAB)
                       + ea1_v[sl] * BOND_VOCAB + ea2_v[sl])
        return 0
    lax.fori_loop(0, EPW // 16, _code, 0)
    pltpu.sync_copy(ecode_v, code_h.at[pl.ds(base_e, EPW)])


def _bond_table_body(bf_ref, bt_ref):
    cv = lax.broadcasted_iota(jnp.int32, (D, 1), 0)
    cols = lax.broadcasted_iota(jnp.int32, (D, D), 1)
    f0 = cv // (BOND_VOCAB * BOND_VOCAB)
    f1 = (cv // BOND_VOCAB) % BOND_VOCAB + BOND_VOCAB
    f2 = cv % BOND_VOCAB + 2 * BOND_VOCAB
    del cols
    t = None
    for fx in (f0, f1, f2):
        row = jnp.zeros((D, D), jnp.float32)
        for k in range(BOND_F * BOND_VOCAB):
            row = jnp.where(fx == k, bf_ref[k:k + 1, :], row)
        t = row if t is None else t + row
    bt_ref[...] = t


def _tc_bond_table(bondf_pad):
    return pl.pallas_call(
        _bond_table_body,
        out_shape=jax.ShapeDtypeStruct((D, D), jnp.float32),
    )(bondf_pad)


@functools.partial(
    pl.kernel,
    out_type=jax.ShapeDtypeStruct((2 * NP, D), jnp.float32),
    mesh=_mesh,
    scratch_types=[
        pltpu.VMEM((KE,), jnp.int32),
        pltpu.VMEM((KE,), jnp.int32),
        pltpu.VMEM((KE,), jnp.int32),
        pltpu.VMEM((KE, D), jnp.float32),
        pltpu.VMEM((D, D), jnp.float32),
        pltpu.VMEM_SHARED((D, D), jnp.float32),
        pltpu.VMEM_SHARED((NP, D), jnp.float32),
        pltpu.SemaphoreType.DMA,
    ],
)
def _sc_message(h_h, code_h, src_h, dst_h, bt_h, agg_h,
                src_v, dst_v, code_v, buf, zbuf, bt_sh, agg_sh, sem):
    c = lax.axis_index("c")
    s = lax.axis_index("s")
    wid = s * NC + c

    @pl.when(s == 0)
    def _():
        pltpu.sync_copy(bt_h, bt_sh)

    def _zero(i, _):
        zbuf[i // 8, pl.ds((i % 8) * 16, 16)] = jnp.zeros((16,), jnp.float32)
        return 0
    lax.fori_loop(0, D * D // 16, _zero, 0)
    for k in range(ROWS_PER_S // D):
        pltpu.sync_copy(zbuf, agg_sh.at[pl.ds(s * ROWS_PER_S + k * D, D)])
    plsc.subcore_barrier()

    base_e = wid * EPW

    def _chunk(t, _):
        off = base_e + t * KE
        pltpu.sync_copy(src_h.at[pl.ds(off, KE)], src_v)
        pltpu.sync_copy(code_h.at[pl.ds(off, KE)], code_v)
        pltpu.sync_copy(dst_h.at[pl.ds(off, KE)], dst_v)
        pltpu.async_copy(bt_sh.at[code_v], buf, sem).wait()
        pltpu.async_copy(h_h.at[src_v], buf, sem, add=True).wait()

        def _row(j, _):
            for g in range(D // 16):
                sl = pl.ds(g * 16, 16)
                buf[j, sl] = jnp.maximum(buf[j, sl], 0.0)
            return 0
        lax.fori_loop(0, KE, _row, 0)
        pltpu.sync_copy(buf, agg_sh.at[dst_v], add=True)
        return 0
    lax.fori_loop(0, NCHK, _chunk, 0)

    plsc.subcore_barrier()
    for k in range(ROWS_PER_S // D):
        r = s * ROWS_PER_S + k * D
        pltpu.sync_copy(agg_sh.at[pl.ds(r, D)],
                        agg_h.at[pl.ds(c * NP + r, D)])


BLK = 1024
NBLK = NP // BLK
EPS = 1e-5


def _rowmask(i):
    rows = lax.broadcasted_iota(jnp.int32, (BLK, 1), 0) + i * BLK
    return (rows < N).astype(jnp.float32)


def _stats_update(x, i, s_ref):
    @pl.when(i == 0)
    def _():
        s_ref[0:1, :] = jnp.mean(x, axis=0, keepdims=True)

    m = _rowmask(i)
    xc = (x - s_ref[0:1, :]) * m
    st = jnp.concatenate([jnp.sum(xc, axis=0, keepdims=True),
                          jnp.sum(xc * xc, axis=0, keepdims=True)], axis=0)

    @pl.when(i == 0)
    def _():
        s_ref[1:3, :] = st

    @pl.when(i > 0)
    def _():
        s_ref[1:3, :] += st


def _passA_body(h_ref, a0_ref, a1_ref, w_ref, b_ref, u_ref, s_ref):
    i = pl.program_id(0)
    z = h_ref[...] + a0_ref[...] + a1_ref[...]
    u = jnp.dot(z, w_ref[...], preferred_element_type=jnp.float32) + b_ref[...]
    u_ref[...] = u
    _stats_update(u, i, s_ref)


def _tc_passA(h, agg, w, b):
    return pl.pallas_call(
        _passA_body,
        grid=(NBLK,),
        in_specs=[
            pl.BlockSpec((BLK, D), lambda i: (i, 0)),
            pl.BlockSpec((BLK, D), lambda i: (i, 0)),
            pl.BlockSpec((BLK, D), lambda i: (i + NBLK, 0)),
            pl.BlockSpec((D, 2 * D), lambda i: (0, 0)),
            pl.BlockSpec((1, 2 * D), lambda i: (0, 0)),
        ],
        out_specs=[
            pl.BlockSpec((BLK, 2 * D), lambda i: (i, 0)),
            pl.BlockSpec((3, 2 * D), lambda i: (0, 0)),
        ],
        out_shape=[
            jax.ShapeDtypeStruct((NP, 2 * D), jnp.float32),
            jax.ShapeDtypeStruct((3, 2 * D), jnp.float32),
        ],
    )(h, agg, agg, w, b)


def _bn_scale(s_ref, st_scr):
    dm = s_ref[1:2, :] * (1.0 / N)
    mean = s_ref[0:1, :] + dm
    var = s_ref[2:3, :] * (1.0 / N) - dm * dm
    st_scr[0:1, :] = mean
    st_scr[1:2, :] = jnp.sqrt(var + EPS)


def _bn_apply(x, g_ref, be_ref, st_scr):
    return (x - st_scr[0:1, :]) / st_scr[1:2, :] * g_ref[...] + be_ref[...]


def _passB_body(u_ref, s_ref, g_ref, be_ref, w_ref, b_ref, v_ref, s2_ref,
                st_scr):
    i = pl.program_id(0)

    @pl.when(i == 0)
    def _():
        _bn_scale(s_ref, st_scr)

    r = jnp.maximum(_bn_apply(u_ref[...], g_ref, be_ref, st_scr), 0.0)
    v = jnp.dot(r, w_ref[...], preferred_element_type=jnp.float32) + b_ref[...]
    v_ref[...] = v
    _stats_update(v, i, s2_ref)


def _tc_passB(u, sums, g, be, w, b):
    return pl.pallas_call(
        _passB_body,
        grid=(NBLK,),
        in_specs=[
            pl.BlockSpec((BLK, 2 * D), lambda i: (i, 0)),
            pl.BlockSpec((3, 2 * D), lambda i: (0, 0)),
            pl.BlockSpec((1, 2 * D), lambda i: (0, 0)),
            pl.BlockSpec((1, 2 * D), lambda i: (0, 0)),
            pl.BlockSpec((2 * D, D), lambda i: (0, 0)),
            pl.BlockSpec((1, D), lambda i: (0, 0)),
        ],
        out_specs=[
            pl.BlockSpec((BLK, D), lambda i: (i, 0)),
            pl.BlockSpec((3, D), lambda i: (0, 0)),
        ],
        out_shape=[
            jax.ShapeDtypeStruct((NP, D), jnp.float32),
            jax.ShapeDtypeStruct((3, D), jnp.float32),
        ],
        scratch_shapes=[pltpu.VMEM((2, 2 * D), jnp.float32)],
    )(u, sums, g, be, w, b)


def _passC_body(v_ref, s_ref, g_ref, be_ref, h_ref, st_scr):
    i = pl.program_id(0)

    @pl.when(i == 0)
    def _():
        _bn_scale(s_ref, st_scr)

    h_ref[...] = jnp.maximum(_bn_apply(v_ref[...], g_ref, be_ref, st_scr), 0.0)


def _tc_passC(v, sums, g, be):
    return pl.pallas_call(
        _passC_body,
        grid=(NBLK,),
        in_specs=[
            pl.BlockSpec((BLK, D), lambda i: (i, 0)),
            pl.BlockSpec((3, D), lambda i: (0, 0)),
            pl.BlockSpec((1, D), lambda i: (0, 0)),
            pl.BlockSpec((1, D), lambda i: (0, 0)),
        ],
        out_specs=pl.BlockSpec((BLK, D), lambda i: (i, 0)),
        out_shape=jax.ShapeDtypeStruct((NP, D), jnp.float32),
        scratch_shapes=[pltpu.VMEM((2, D), jnp.float32)],
    )(v, sums, g, be)


def _passC_last_body(v_ref, s_ref, g_ref, be_ref, batch_ref, h_ref, xp_ref,
                     st_scr):
    i = pl.program_id(0)

    @pl.when(i == 0)
    def _():
        _bn_scale(s_ref, st_scr)

    hv = _bn_apply(v_ref[...], g_ref, be_ref, st_scr)
    h_ref[...] = hv
    gids = lax.broadcasted_iota(jnp.int32, (BLK, NG), 1)
    oh = (batch_ref[...] == gids).astype(jnp.float32) * _rowmask(i)
    part = lax.dot_general(oh, hv, (((0,), (0,)), ((), ())),
                           preferred_element_type=jnp.float32, precision=lax.Precision.HIGHEST)

    @pl.when(i == 0)
    def _():
        xp_ref[...] = part

    @pl.when(i > 0)
    def _():
        xp_ref[...] += part


def _tc_passC_last(v, sums, g, be, batch_p):
    return pl.pallas_call(
        _passC_last_body,
        grid=(NBLK,),
        in_specs=[
            pl.BlockSpec((BLK, D), lambda i: (i, 0)),
            pl.BlockSpec((3, D), lambda i: (0, 0)),
            pl.BlockSpec((1, D), lambda i: (0, 0)),
            pl.BlockSpec((1, D), lambda i: (0, 0)),
            pl.BlockSpec((BLK, 1), lambda i: (i, 0)),
        ],
        out_specs=[
            pl.BlockSpec((BLK, D), lambda i: (i, 0)),
            pl.BlockSpec((NG, D), lambda i: (0, 0)),
        ],
        out_shape=[
            jax.ShapeDtypeStruct((NP, D), jnp.float32),
            jax.ShapeDtypeStruct((NG, D), jnp.float32),
        ],
        scratch_shapes=[pltpu.VMEM((2, D), jnp.float32)],
    )(v, sums, g, be, batch_p)


def kernel(batch, x, edge_index, edge_attr, atom_emb, bond_emb,
           W1, b1, gamma1, beta1, W2, b2, gamma2, beta2):
    x = x.astype(jnp.int32)
    edge_index = edge_index.astype(jnp.int32)
    edge_attr = edge_attr.astype(jnp.int32)
    batch = batch.astype(jnp.int32)

    xT = jnp.pad(x, ((0, NP - N), (0, 0))).T.reshape(ATOM_F * NP)
    eaT = jnp.pad(edge_attr, ((0, EP - E), (0, 0))).T.reshape(BOND_F * EP)
    src = jnp.pad(edge_index[0], (0, EP - E))
    dst = jnp.pad(edge_index[1], (0, EP - E), constant_values=N)
    atomf = atom_emb.reshape(ATOM_F * ATOM_VOCAB, D)
    bondf = bond_emb.reshape(BOND_F * BOND_VOCAB, D)
    batch_p = jnp.pad(batch, (0, NP - N)).reshape(NP, 1)

    h, code = _sc_encode(xT, eaT, atomf)
    btable = _tc_bond_table(jnp.pad(bondf, ((0, D - BOND_F * BOND_VOCAB),
                                            (0, 0))))

    xpool = None
    for i in range(L):
        agg = _sc_message(h, code, src, dst, btable)
        u, sums1 = _tc_passA(h, agg, W1[i], b1[i].reshape(1, 2 * D))
        v, sums2 = _tc_passB(u, sums1, gamma1[i].reshape(1, 2 * D),
                             beta1[i].reshape(1, 2 * D), W2[i],
                             b2[i].reshape(1, D))
        if i < L - 1:
            h = _tc_passC(v, sums2, gamma2[i].reshape(1, D),
                          beta2[i].reshape(1, D))
        else:
            h, xpool = _tc_passC_last(v, sums2, gamma2[i].reshape(1, D),
                                      beta2[i].reshape(1, D), batch_p)

    return xpool, h[:N]
